# baseline (speedup 1.0000x reference)
import functools

import jax
import jax.numpy as jnp
from jax import lax
from jax.experimental import pallas as pl
from jax.experimental.pallas import tpu as pltpu
from jax.experimental.pallas import tpu_sc as plsc

N = 1_600_000
SEG = 1024
NF = 8

_info = plsc.get_sparse_core_info()
NC, NS, L = _info.num_cores, _info.num_subcores, _info.num_lanes
NW = NC * NS
PER_W = N // NW
CHUNK = 2000
NCHUNK = PER_W // CHUNK
GROUPS = CHUNK // 16
UNROLL = 2
WINDOWS = GROUPS // UNROLL

_mesh = plsc.VectorSubcoreMesh(core_axis_name="c", subcore_axis_name="s")


@functools.partial(
    pl.kernel,
    out_type=jax.ShapeDtypeStruct((NW, NF * SEG), jnp.float32),
    mesh=_mesh,
    compiler_params=pltpu.CompilerParams(needs_layout_passes=False),
    scratch_types=[
        pltpu.VMEM((CHUNK,), jnp.float32),
        pltpu.VMEM((CHUNK,), jnp.float32),
        pltpu.VMEM((CHUNK,), jnp.int32),
        pltpu.VMEM((CHUNK,), jnp.int32),
        pltpu.VMEM((CHUNK,), jnp.int32),
        pltpu.VMEM((CHUNK,), jnp.int32),
        pltpu.VMEM((CHUNK * 3,), jnp.float32),
        pltpu.VMEM((CHUNK * 3,), jnp.float32),
        pltpu.VMEM((NF * SEG,), jnp.float32),
        pltpu.SemaphoreType.DMA,
        pltpu.SemaphoreType.DMA,
    ],
)
def _sc_partials(pred_h, targ_h, batch_h, posf_h, out_h,
                 pred_b0, pred_b1, targ_b0, targ_b1, batch_b0, batch_b1,
                 pos_b0, pos_b1, acc, sem0, sem1):
    wid = lax.axis_index("s") * NC + lax.axis_index("c")
    base = wid * PER_W
    sems = (sem0, sem1)
    bufs = (
        (pred_b0, targ_b0, batch_b0, pos_b0),
        (pred_b1, targ_b1, batch_b1, pos_b1),
    )

    zero16 = jnp.zeros((16,), jnp.float32)

    def zbody(i, carry):
        acc[pl.ds(i * 16, 16)] = zero16
        return carry

    lax.fori_loop(0, NF * SEG // 16, zbody, 0)

    def start_chunk(k, slot):
        off = base + k * CHUNK
        sem = sems[slot]
        pb, tb, bb, ob = bufs[slot]
        pltpu.make_async_copy(pred_h.at[pl.ds(off, CHUNK)], pb, sem).start()
        pltpu.make_async_copy(targ_h.at[pl.ds(off, CHUNK)], tb, sem).start()
        pltpu.make_async_copy(batch_h.at[pl.ds(off, CHUNK)], bb, sem).start()
        pltpu.make_async_copy(posf_h.at[pl.ds(off * 3, CHUNK * 3)], ob, sem).start()

    def wait_chunk(k, slot):
        off = base + k * CHUNK
        sem = sems[slot]
        pb, tb, bb, ob = bufs[slot]
        pltpu.make_async_copy(pred_h.at[pl.ds(off, CHUNK)], pb, sem).wait()
        pltpu.make_async_copy(targ_h.at[pl.ds(off, CHUNK)], tb, sem).wait()
        pltpu.make_async_copy(batch_h.at[pl.ds(off, CHUNK)], bb, sem).wait()
        pltpu.make_async_copy(posf_h.at[pl.ds(off * 3, CHUNK * 3)], ob, sem).wait()

    iota = lax.iota(jnp.int32, 16)
    iota3 = iota * 3
    zerov = jnp.zeros((16,), jnp.float32)
    zeroi = jnp.zeros((16,), jnp.int32)
    lane15 = iota == 15

    def flush(cur, cs):
        for f in range(NF):
            tot = plsc.cumsum(cs[f])
            idxv = zeroi + (f * SEG + cur)
            plsc.addupdate_scatter(acc, [idxv], tot, mask=lane15)

    def make_window(slot, u_groups):
        pb, tb, bb, ob = bufs[slot]

        def window(o, carry):
            cur = carry[0]
            cs = carry[1:]
            bvs = [bb[pl.ds(o + 16 * u, 16)] for u in range(u_groups)]
            bfirst = bvs[0][0]
            blast = bvs[-1][15]
            uniform = bfirst == blast
            same = jnp.logical_and(uniform, bfirst == cur)

            qs = []
            for u in range(u_groups):
                ou = o + 16 * u
                p = pb[pl.ds(ou, 16)]
                t = tb[pl.ds(ou, 16)]
                ox = ou * 3
                x = plsc.load_gather(ob, [ox + iota3])
                y = plsc.load_gather(ob, [ox + iota3 + 1])
                z = plsc.load_gather(ob, [ox + iota3 + 2])
                wt = t.astype(jnp.float32)
                qs.append((wt * x, wt * y, wt * z, wt, p * x, p * y, p * z, p))
            contribs = tuple(
                functools.reduce(lambda a_, b_: a_ + b_, [qs[u][f] for u in range(u_groups)])
                for f in range(NF)
            )

            def case_same():
                return (cur,) + tuple(c + d for c, d in zip(cs, contribs))

            def case_flush():
                flush(cur, cs)

                def case_new():
                    return (bfirst,) + contribs

                def case_scatter():
                    for u in range(u_groups):
                        for f in range(NF):
                            plsc.addupdate_scatter(acc, [bvs[u] + f * SEG], qs[u][f])
                    return (blast,) + (zerov,) * NF

                return lax.cond(uniform, case_new, case_scatter)

            return lax.cond(same, case_same, case_flush)

        return window

    win2 = (make_window(0, UNROLL), make_window(1, UNROLL))
    win1 = (make_window(0, 1), make_window(1, 1))

    carry = (jnp.int32(0),) + (zerov,) * NF
    start_chunk(0, 0)
    for k in range(NCHUNK):
        slot = k % 2
        if k + 1 < NCHUNK:
            start_chunk(k + 1, (k + 1) % 2)
        wait_chunk(k, slot)
        wfn = win2[slot]
        carry = lax.fori_loop(
            0, WINDOWS, lambda w, c, _f=wfn: _f(w * (16 * UNROLL), c), carry)
        carry = win1[slot](WINDOWS * UNROLL * 16, carry)
    flush(carry[0], carry[1:])

    pltpu.sync_copy(acc, out_h.at[wid])


def _finish_body(p_ref, o_ref):
    a = jnp.sum(p_ref[...], axis=0)
    eps = jnp.float32(1e-10)
    tc = a[0:3, :] / (a[3:4, :] + eps)
    pc = a[4:7, :] / (a[7:8, :] + eps)
    d = tc - pc
    o_ref[0, 0] = jnp.sqrt(jnp.sum(d * d))


_finish = pl.pallas_call(
    _finish_body,
    out_shape=jax.ShapeDtypeStruct((1, 1), jnp.float32),
    out_specs=pl.BlockSpec(memory_space=pltpu.SMEM),
)


def kernel(pred, target, batch, pos):
    posf = pos.reshape(-1)
    partials = _sc_partials(pred, target, batch, posf)
    loss = _finish(partials.reshape(NW, NF, SEG))
    return loss[0, 0]


# speedup vs baseline: 33.7191x; 33.7191x over previous
import functools

import jax
import jax.numpy as jnp
from jax import lax
from jax.experimental import pallas as pl
from jax.experimental.pallas import tpu as pltpu
from jax.experimental.pallas import tpu_sc as plsc

N = 1_600_000
SEG = 1024
NF = 8

_info = plsc.get_sparse_core_info()
NC, NS, L = _info.num_cores, _info.num_subcores, _info.num_lanes
NW = NC * NS
PER_W = N // NW
CHUNK = 2000
NCHUNK = PER_W // CHUNK
GROUPS = CHUNK // 16
UNROLL = 2
WINDOWS = GROUPS // UNROLL

_mesh = plsc.VectorSubcoreMesh(core_axis_name="c", subcore_axis_name="s")


@functools.partial(
    pl.kernel,
    out_type=jax.ShapeDtypeStruct((NW, NF * SEG), jnp.float32),
    mesh=_mesh,
    compiler_params=pltpu.CompilerParams(needs_layout_passes=False),
    scratch_types=[
        pltpu.VMEM((CHUNK,), jnp.float32),
        pltpu.VMEM((CHUNK,), jnp.float32),
        pltpu.VMEM((CHUNK,), jnp.int32),
        pltpu.VMEM((CHUNK,), jnp.int32),
        pltpu.VMEM((CHUNK,), jnp.int32),
        pltpu.VMEM((CHUNK,), jnp.int32),
        pltpu.VMEM((CHUNK,), jnp.float32),
        pltpu.VMEM((CHUNK,), jnp.float32),
        pltpu.VMEM((CHUNK,), jnp.float32),
        pltpu.VMEM((CHUNK,), jnp.float32),
        pltpu.VMEM((CHUNK,), jnp.float32),
        pltpu.VMEM((CHUNK,), jnp.float32),
        pltpu.VMEM((NF * SEG,), jnp.float32),
        pltpu.SemaphoreType.DMA,
        pltpu.SemaphoreType.DMA,
    ],
)
def _sc_partials(pred_h, targ_h, batch_h, px_h, py_h, pz_h, out_h,
                 pred_b0, pred_b1, targ_b0, targ_b1, batch_b0, batch_b1,
                 px_b0, px_b1, py_b0, py_b1, pz_b0, pz_b1, acc, sem0, sem1):
    wid = lax.axis_index("s") * NC + lax.axis_index("c")
    base = wid * PER_W
    sems = (sem0, sem1)
    bufs = (
        (pred_b0, targ_b0, batch_b0, px_b0, py_b0, pz_b0),
        (pred_b1, targ_b1, batch_b1, px_b1, py_b1, pz_b1),
    )
    hbm = (pred_h, targ_h, batch_h, px_h, py_h, pz_h)

    zero16 = jnp.zeros((16,), jnp.float32)

    def zbody(i, carry):
        acc[pl.ds(i * 16, 16)] = zero16
        return carry

    lax.fori_loop(0, NF * SEG // 16, zbody, 0)

    def start_chunk(k, slot):
        off = base + k * CHUNK
        sem = sems[slot]
        for h, b in zip(hbm, bufs[slot]):
            pltpu.make_async_copy(h.at[pl.ds(off, CHUNK)], b, sem).start()

    def wait_chunk(k, slot):
        off = base + k * CHUNK
        sem = sems[slot]
        for h, b in zip(hbm, bufs[slot]):
            pltpu.make_async_copy(h.at[pl.ds(off, CHUNK)], b, sem).wait()

    iota = lax.iota(jnp.int32, 16)
    zerov = jnp.zeros((16,), jnp.float32)
    zeroi = jnp.zeros((16,), jnp.int32)
    lane15 = iota == 15

    def flush(cur, cs):
        for f in range(NF):
            tot = plsc.cumsum(cs[f])
            idxv = zeroi + (f * SEG + cur)
            plsc.addupdate_scatter(acc, [idxv], tot, mask=lane15)

    def make_window(slot, u_groups):
        pb, tb, bb, xb, yb, zb = bufs[slot]

        def window(o, carry):
            cur = carry[0]
            cs = carry[1:]
            bvs = [bb[pl.ds(o + 16 * u, 16)] for u in range(u_groups)]
            bfirst = bvs[0][0]
            blast = bvs[-1][15]
            uniform = bfirst == blast
            same = jnp.logical_and(uniform, bfirst == cur)

            qs = []
            for u in range(u_groups):
                ou = o + 16 * u
                p = pb[pl.ds(ou, 16)]
                t = tb[pl.ds(ou, 16)]
                x = xb[pl.ds(ou, 16)]
                y = yb[pl.ds(ou, 16)]
                z = zb[pl.ds(ou, 16)]
                wt = t.astype(jnp.float32)
                qs.append((wt * x, wt * y, wt * z, wt, p * x, p * y, p * z, p))
            contribs = tuple(
                functools.reduce(lambda a_, b_: a_ + b_, [qs[u][f] for u in range(u_groups)])
                for f in range(NF)
            )

            def case_same():
                return (cur,) + tuple(c + d for c, d in zip(cs, contribs))

            def case_flush():
                flush(cur, cs)

                def case_new():
                    return (bfirst,) + contribs

                def case_scatter():
                    for u in range(u_groups):
                        for f in range(NF):
                            plsc.addupdate_scatter(acc, [bvs[u] + f * SEG], qs[u][f])
                    return (blast,) + (zerov,) * NF

                return lax.cond(uniform, case_new, case_scatter)

            return lax.cond(same, case_same, case_flush)

        return window

    win2 = (make_window(0, UNROLL), make_window(1, UNROLL))
    win1 = (make_window(0, 1), make_window(1, 1))

    carry = (jnp.int32(0),) + (zerov,) * NF
    start_chunk(0, 0)
    for k in range(NCHUNK):
        slot = k % 2
        if k + 1 < NCHUNK:
            start_chunk(k + 1, (k + 1) % 2)
        wait_chunk(k, slot)
        wfn = win2[slot]
        carry = lax.fori_loop(
            0, WINDOWS, lambda w, c, _f=wfn: _f(w * (16 * UNROLL), c), carry)
        carry = win1[slot](WINDOWS * UNROLL * 16, carry)
    flush(carry[0], carry[1:])

    pltpu.sync_copy(acc, out_h.at[wid])


def _finish_body(p_ref, o_ref):
    a = jnp.sum(p_ref[...], axis=0)
    eps = jnp.float32(1e-10)
    tc = a[0:3, :] / (a[3:4, :] + eps)
    pc = a[4:7, :] / (a[7:8, :] + eps)
    d = tc - pc
    o_ref[0, 0] = jnp.sqrt(jnp.sum(d * d))


_finish = pl.pallas_call(
    _finish_body,
    out_shape=jax.ShapeDtypeStruct((1, 1), jnp.float32),
    out_specs=pl.BlockSpec(memory_space=pltpu.SMEM),
)


def kernel(pred, target, batch, pos):
    partials = _sc_partials(pred, target, batch,
                            pos[:, 0], pos[:, 1], pos[:, 2])
    loss = _finish(partials.reshape(NW, NF, SEG))
    return loss[0, 0]


# speedup vs baseline: 35.5648x; 1.0547x over previous
import functools

import jax
import jax.numpy as jnp
from jax import lax
from jax.experimental import pallas as pl
from jax.experimental.pallas import tpu as pltpu
from jax.experimental.pallas import tpu_sc as plsc

N = 1_600_000
SEG = 1024
NF = 8

_info = plsc.get_sparse_core_info()
NC, NS, L = _info.num_cores, _info.num_subcores, _info.num_lanes
NW = NC * NS
PER_W = N // NW
CHUNK = 2000
NCHUNK = PER_W // CHUNK
GROUPS = CHUNK // 16
UNROLL = 4
WINDOWS = GROUPS // UNROLL

_mesh = plsc.VectorSubcoreMesh(core_axis_name="c", subcore_axis_name="s")


@functools.partial(
    pl.kernel,
    out_type=jax.ShapeDtypeStruct((NW, NF * SEG), jnp.float32),
    mesh=_mesh,
    compiler_params=pltpu.CompilerParams(needs_layout_passes=False),
    scratch_types=[
        pltpu.VMEM((CHUNK,), jnp.float32),
        pltpu.VMEM((CHUNK,), jnp.float32),
        pltpu.VMEM((CHUNK,), jnp.int32),
        pltpu.VMEM((CHUNK,), jnp.int32),
        pltpu.VMEM((CHUNK,), jnp.int32),
        pltpu.VMEM((CHUNK,), jnp.int32),
        pltpu.VMEM((CHUNK,), jnp.float32),
        pltpu.VMEM((CHUNK,), jnp.float32),
        pltpu.VMEM((CHUNK,), jnp.float32),
        pltpu.VMEM((CHUNK,), jnp.float32),
        pltpu.VMEM((CHUNK,), jnp.float32),
        pltpu.VMEM((CHUNK,), jnp.float32),
        pltpu.VMEM((NF * SEG,), jnp.float32),
        pltpu.SemaphoreType.DMA,
        pltpu.SemaphoreType.DMA,
    ],
)
def _sc_partials(pred_h, targ_h, batch_h, px_h, py_h, pz_h, out_h,
                 pred_b0, pred_b1, targ_b0, targ_b1, batch_b0, batch_b1,
                 px_b0, px_b1, py_b0, py_b1, pz_b0, pz_b1, acc, sem0, sem1):
    wid = lax.axis_index("s") * NC + lax.axis_index("c")
    base = wid * PER_W
    sems = (sem0, sem1)
    bufs = (
        (pred_b0, targ_b0, batch_b0, px_b0, py_b0, pz_b0),
        (pred_b1, targ_b1, batch_b1, px_b1, py_b1, pz_b1),
    )
    hbm = (pred_h, targ_h, batch_h, px_h, py_h, pz_h)

    zero16 = jnp.zeros((16,), jnp.float32)

    def zbody(i, carry):
        acc[pl.ds(i * 16, 16)] = zero16
        return carry

    lax.fori_loop(0, NF * SEG // 16, zbody, 0)

    def start_chunk(k, slot):
        off = base + k * CHUNK
        sem = sems[slot]
        for h, b in zip(hbm, bufs[slot]):
            pltpu.make_async_copy(h.at[pl.ds(off, CHUNK)], b, sem).start()

    def wait_chunk(k, slot):
        off = base + k * CHUNK
        sem = sems[slot]
        for h, b in zip(hbm, bufs[slot]):
            pltpu.make_async_copy(h.at[pl.ds(off, CHUNK)], b, sem).wait()

    iota = lax.iota(jnp.int32, 16)
    zerov = jnp.zeros((16,), jnp.float32)
    zeroi = jnp.zeros((16,), jnp.int32)
    lane15 = iota == 15

    def flush(cur, cs):
        for f in range(NF):
            tot = plsc.cumsum(cs[f])
            idxv = zeroi + (f * SEG + cur)
            plsc.addupdate_scatter(acc, [idxv], tot, mask=lane15)

    def make_window(slot, u_groups):
        pb, tb, bb, xb, yb, zb = bufs[slot]

        def window(o, carry):
            cur = carry[0]
            cs = carry[1:]
            bvs = [bb[pl.ds(o + 16 * u, 16)] for u in range(u_groups)]
            bfirst = bvs[0][0]
            blast = bvs[-1][15]
            uniform = bfirst == blast
            same = jnp.logical_and(uniform, bfirst == cur)

            qs = []
            for u in range(u_groups):
                ou = o + 16 * u
                p = pb[pl.ds(ou, 16)]
                t = tb[pl.ds(ou, 16)]
                x = xb[pl.ds(ou, 16)]
                y = yb[pl.ds(ou, 16)]
                z = zb[pl.ds(ou, 16)]
                wt = t.astype(jnp.float32)
                qs.append((wt * x, wt * y, wt * z, wt, p * x, p * y, p * z, p))
            contribs = tuple(
                functools.reduce(lambda a_, b_: a_ + b_, [qs[u][f] for u in range(u_groups)])
                for f in range(NF)
            )

            def case_same():
                return (cur,) + tuple(c + d for c, d in zip(cs, contribs))

            def case_flush():
                flush(cur, cs)

                def case_new():
                    return (bfirst,) + contribs

                def case_scatter():
                    for u in range(u_groups):
                        for f in range(NF):
                            plsc.addupdate_scatter(acc, [bvs[u] + f * SEG], qs[u][f])
                    return (blast,) + (zerov,) * NF

                return lax.cond(uniform, case_new, case_scatter)

            return lax.cond(same, case_same, case_flush)

        return window

    win2 = (make_window(0, UNROLL), make_window(1, UNROLL))
    win1 = (make_window(0, 1), make_window(1, 1))

    carry = (jnp.int32(0),) + (zerov,) * NF
    start_chunk(0, 0)
    for k in range(NCHUNK):
        slot = k % 2
        if k + 1 < NCHUNK:
            start_chunk(k + 1, (k + 1) % 2)
        wait_chunk(k, slot)
        wfn = win2[slot]
        carry = lax.fori_loop(
            0, WINDOWS, lambda w, c, _f=wfn: _f(w * (16 * UNROLL), c), carry)
        carry = win1[slot](WINDOWS * UNROLL * 16, carry)
    flush(carry[0], carry[1:])

    pltpu.sync_copy(acc, out_h.at[wid])


def _finish_body(p_ref, o_ref):
    a = jnp.sum(p_ref[...], axis=0)
    eps = jnp.float32(1e-10)
    tc = a[0:3, :] / (a[3:4, :] + eps)
    pc = a[4:7, :] / (a[7:8, :] + eps)
    d = tc - pc
    o_ref[0, 0] = jnp.sqrt(jnp.sum(d * d))


_finish = pl.pallas_call(
    _finish_body,
    out_shape=jax.ShapeDtypeStruct((1, 1), jnp.float32),
    out_specs=pl.BlockSpec(memory_space=pltpu.SMEM),
)


def kernel(pred, target, batch, pos):
    partials = _sc_partials(pred, target, batch,
                            pos[:, 0], pos[:, 1], pos[:, 2])
    loss = _finish(partials.reshape(NW, NF, SEG))
    return loss[0, 0]
